# Initial kernel scaffold; baseline (speedup 1.0000x reference)
import jax
import jax.numpy as jnp
from jax.experimental import pallas as pl


def kernel(feats, edge_index, W0, b0, gamma0, beta0, W1, b1):
    raise NotImplementedError("write your pallas kernel here")


# speedup vs baseline: 8.5920x; 8.5920x over previous
import dataclasses
import functools

import jax
import jax.numpy as jnp
from jax import lax
from jax.experimental import pallas as pl
from jax.experimental.pallas import tpu as pltpu
from jax.experimental.pallas import tpu_sc as plsc

N = 10000
E = 320000
D = 128
NP = 10240

NC = 2
NS = 16
NW = NC * NS
EW = E // NW
C = 40
G = 25
NBLK = EW // (G * C)
HR = NP // D


def _sc_segsum_body(y_hbm, src_hbm, dst_hbm, out_hbm,
                    srcb, dstb, rows0, rows1, acc_sh, semA, semB):
    cid = lax.axis_index("c")
    sid = lax.axis_index("s")
    wid = sid * NC + cid

    zrow = jnp.zeros((16,), jnp.float32)

    @pl.loop(0, C)
    def _(i):
        @pl.loop(0, D // 16)
        def _(j):
            rows0[i, pl.ds(j * 16, 16)] = zrow

    @pl.loop(sid, N // C, step=NS)
    def _(k):
        pltpu.sync_copy(rows0, acc_sh.at[pl.ds(k * C, C)])

    plsc.subcore_barrier()

    def gather(i, buf, sem):
        return pltpu.async_copy(y_hbm.at[srcb.at[i]], buf, sem)

    def gwait(i, buf, sem):
        pltpu.make_async_copy(y_hbm.at[srcb.at[i]], buf, sem).wait()

    def scat(i, buf):
        pltpu.sync_copy(buf, acc_sh.at[dstb.at[i]], add=True)

    @pl.loop(0, NBLK)
    def _(b):
        pltpu.sync_copy(src_hbm.at[wid, b], srcb)
        pltpu.sync_copy(dst_hbm.at[wid, b], dstb)
        gather(0, rows0, semA)

        @pl.loop(0, G - 1, step=2)
        def _(j):
            gather(j + 1, rows1, semB)
            gwait(j, rows0, semA)
            scat(j, rows0)
            gather(j + 2, rows0, semA)
            gwait(j + 1, rows1, semB)
            scat(j + 1, rows1)

        gwait(G - 1, rows0, semA)
        scat(G - 1, rows0)

    plsc.subcore_barrier()

    @pl.loop(sid, N // C, step=NS)
    def _(k):
        pltpu.sync_copy(acc_sh.at[pl.ds(k * C, C)],
                        out_hbm.at[cid, pl.ds(k * C, C)])


@functools.cache
def _make_sc_segsum():
    mesh = plsc.VectorSubcoreMesh(core_axis_name="c", subcore_axis_name="s")
    return pl.kernel(
        _sc_segsum_body,
        out_type=jax.ShapeDtypeStruct((NC, NP, D), jnp.float32),
        mesh=mesh,
        scratch_types=[
            pltpu.VMEM((G, C), jnp.int32),
            pltpu.VMEM((G, C), jnp.int32),
            pltpu.VMEM((C, D), jnp.float32),
            pltpu.VMEM((C, D), jnp.float32),
            pltpu.VMEM_SHARED((N, D), jnp.float32),
            pltpu.SemaphoreType.DMA,
            pltpu.SemaphoreType.DMA,
        ],
    )


def _sc_deg_body(dst_hbm, out_hbm, dstb, hist, idb, acc_sh):
    cid = lax.axis_index("c")
    sid = lax.axis_index("s")
    wid = sid * NC + cid

    zrow = jnp.zeros((16,), jnp.float32)
    ones = jnp.ones((16,), jnp.float32)
    iota = lax.iota(jnp.int32, 16)

    @pl.loop(0, HR)
    def _(i):
        @pl.loop(0, D // 16)
        def _(j):
            hist[i, pl.ds(j * 16, 16)] = zrow

    @pl.loop(0, HR // 16)
    def _(k):
        idb[0, pl.ds(k * 16, 16)] = iota + k * 16

    @pl.loop(sid, HR // 8, step=NS)
    def _(k):
        pltpu.sync_copy(hist.at[pl.ds(0, 8)], acc_sh.at[pl.ds(k * 8, 8)])

    pltpu.sync_copy(dst_hbm.at[wid], dstb)

    plsc.subcore_barrier()

    @pl.loop(0, EW // 16)
    def _(j):
        v = dstb[j]
        hi = lax.shift_right_logical(v, 7)
        lo = lax.bitwise_and(v, 127)
        plsc.addupdate_scatter(hist, [hi, lo], ones)

    pltpu.sync_copy(hist, acc_sh.at[idb.at[0]], add=True)

    plsc.subcore_barrier()

    @pl.loop(sid, HR // 8, step=NS)
    def _(k):
        pltpu.sync_copy(acc_sh.at[pl.ds(k * 8, 8)],
                        out_hbm.at[cid, pl.ds(k * 8, 8)])


@functools.cache
def _make_sc_deg():
    mesh = plsc.VectorSubcoreMesh(core_axis_name="c", subcore_axis_name="s")
    cp = pltpu.CompilerParams()
    if "needs_layout_passes" in pltpu.CompilerParams.__dataclass_fields__:
        cp = dataclasses.replace(cp, needs_layout_passes=False)
    return pl.kernel(
        _sc_deg_body,
        compiler_params=cp,
        out_type=jax.ShapeDtypeStruct((NC, HR, D), jnp.float32),
        mesh=mesh,
        scratch_types=[
            pltpu.VMEM((EW // 16, 16), jnp.int32),
            pltpu.VMEM((HR, D), jnp.float32),
            pltpu.VMEM((1, HR), jnp.int32),
            pltpu.VMEM_SHARED((HR, D), jnp.float32),
        ],
    )


ROWS_BLK = 1024


def _mm_body(x_ref, w_ref, o_ref):
    o_ref[...] = jnp.dot(x_ref[...], w_ref[...],
                         preferred_element_type=jnp.float32)


def _tc_matmul(x, w):
    return pl.pallas_call(
        _mm_body,
        grid=(NP // ROWS_BLK,),
        in_specs=[
            pl.BlockSpec((ROWS_BLK, D), lambda i: (i, 0)),
            pl.BlockSpec((D, D), lambda i: (0, 0)),
        ],
        out_specs=pl.BlockSpec((ROWS_BLK, D), lambda i: (i, 0)),
        out_shape=jax.ShapeDtypeStruct((NP, D), jnp.float32),
    )(x, w)


def _fuse1_body(s_ref, degp_ref, y_ref, b_ref, g_ref, be_ref, w_ref, o_ref):
    s = s_ref[0] + s_ref[1] + y_ref[...]
    deg = degp_ref[0] + degp_ref[1]
    h = s / (deg + 1.0) + b_ref[...]
    mean = jnp.mean(h, axis=-1, keepdims=True)
    var = jnp.mean((h - mean) ** 2, axis=-1, keepdims=True)
    h = (h - mean) * lax.rsqrt(var + 1e-5) * g_ref[...] + be_ref[...]
    h = jnp.maximum(h, 0.0)
    o_ref[...] = jnp.dot(h, w_ref[...], preferred_element_type=jnp.float32)


def _tc_fuse1(s0, degp, y0, b0, gamma0, beta0, W1):
    return pl.pallas_call(
        _fuse1_body,
        grid=(NP // ROWS_BLK,),
        in_specs=[
            pl.BlockSpec((NC, ROWS_BLK, D), lambda i: (0, i, 0)),
            pl.BlockSpec((NC, ROWS_BLK, 1), lambda i: (0, i, 0)),
            pl.BlockSpec((ROWS_BLK, D), lambda i: (i, 0)),
            pl.BlockSpec((1, D), lambda i: (0, 0)),
            pl.BlockSpec((1, D), lambda i: (0, 0)),
            pl.BlockSpec((1, D), lambda i: (0, 0)),
            pl.BlockSpec((D, D), lambda i: (0, 0)),
        ],
        out_specs=pl.BlockSpec((ROWS_BLK, D), lambda i: (i, 0)),
        out_shape=jax.ShapeDtypeStruct((NP, D), jnp.float32),
    )(s0, degp, y0, b0.reshape(1, D), gamma0.reshape(1, D),
      beta0.reshape(1, D), W1)


def _fuse2_body(s_ref, degp_ref, y_ref, b_ref, o_ref):
    s = s_ref[0] + s_ref[1] + y_ref[...]
    deg = degp_ref[0] + degp_ref[1]
    o_ref[...] = s / (deg + 1.0) + b_ref[...]


def _tc_fuse2(s1, degp, y1, b1):
    return pl.pallas_call(
        _fuse2_body,
        grid=(NP // ROWS_BLK,),
        in_specs=[
            pl.BlockSpec((NC, ROWS_BLK, D), lambda i: (0, i, 0)),
            pl.BlockSpec((NC, ROWS_BLK, 1), lambda i: (0, i, 0)),
            pl.BlockSpec((ROWS_BLK, D), lambda i: (i, 0)),
            pl.BlockSpec((1, D), lambda i: (0, 0)),
        ],
        out_specs=pl.BlockSpec((ROWS_BLK, D), lambda i: (i, 0)),
        out_shape=jax.ShapeDtypeStruct((NP, D), jnp.float32),
    )(s1, degp, y1, b1.reshape(1, D))


def kernel(feats, edge_index, W0, b0, gamma0, beta0, W1, b1):
    src = edge_index[0].reshape(NW, NBLK, G, C)
    dst = edge_index[1].reshape(NW, NBLK, G, C)
    dst16 = edge_index[1].reshape(NW, EW // 16, 16)
    featsp = jnp.concatenate(
        [feats, jnp.zeros((NP - N, D), jnp.float32)], axis=0)

    degp = _make_sc_deg()(dst16)
    degp = degp.reshape(NC, NP, 1)
    y0 = _tc_matmul(featsp, W0)
    s0 = _make_sc_segsum()(y0, src, dst)
    y1 = _tc_fuse1(s0, degp, y0, b0, gamma0, beta0, W1)
    s1 = _make_sc_segsum()(y1, src, dst)
    return _tc_fuse2(s1, degp, y1, b1)[:N]


# speedup vs baseline: 11.3292x; 1.3186x over previous
import dataclasses
import functools

import jax
import jax.numpy as jnp
from jax import lax
from jax.experimental import pallas as pl
from jax.experimental.pallas import tpu as pltpu
from jax.experimental.pallas import tpu_sc as plsc

N = 10000
E = 320000
D = 128
NP = 10240

NC = 2
NS = 16
NW = NC * NS
EW = E // NW
C = 80
G = 25
NBLK = EW // (G * C)
HR = NP // D


def _sc_segsum_body(y_hbm, src_hbm, dst_hbm, out_hbm,
                    srcb, dstb, rows0, rows1, acc_sh, semA, semB):
    cid = lax.axis_index("c")
    sid = lax.axis_index("s")
    wid = sid * NC + cid

    zrow = jnp.zeros((16,), jnp.float32)

    @pl.loop(0, C)
    def _(i):
        @pl.loop(0, D // 16)
        def _(j):
            rows0[i, pl.ds(j * 16, 16)] = zrow

    @pl.loop(sid, N // C, step=NS)
    def _(k):
        pltpu.sync_copy(rows0, acc_sh.at[pl.ds(k * C, C)])

    plsc.subcore_barrier()

    def gather(i, buf, sem):
        return pltpu.async_copy(y_hbm.at[srcb.at[i]], buf, sem)

    def gwait(i, buf, sem):
        pltpu.make_async_copy(y_hbm.at[srcb.at[i]], buf, sem).wait()

    def scat(i, buf):
        pltpu.sync_copy(buf, acc_sh.at[dstb.at[i]], add=True)

    @pl.loop(0, NBLK)
    def _(b):
        pltpu.sync_copy(src_hbm.at[wid, b], srcb)
        pltpu.sync_copy(dst_hbm.at[wid, b], dstb)
        gather(0, rows0, semA)

        @pl.loop(0, G - 1, step=2)
        def _(j):
            gather(j + 1, rows1, semB)
            gwait(j, rows0, semA)
            scat(j, rows0)
            gather(j + 2, rows0, semA)
            gwait(j + 1, rows1, semB)
            scat(j + 1, rows1)

        gwait(G - 1, rows0, semA)
        scat(G - 1, rows0)

    plsc.subcore_barrier()

    @pl.loop(sid, N // C, step=NS)
    def _(k):
        pltpu.sync_copy(acc_sh.at[pl.ds(k * C, C)],
                        out_hbm.at[cid, pl.ds(k * C, C)])


@functools.cache
def _make_sc_segsum():
    mesh = plsc.VectorSubcoreMesh(core_axis_name="c", subcore_axis_name="s")
    cp = pltpu.CompilerParams()
    if "needs_layout_passes" in pltpu.CompilerParams.__dataclass_fields__:
        cp = dataclasses.replace(cp, needs_layout_passes=False)
    return pl.kernel(
        _sc_segsum_body,
        compiler_params=cp,
        out_type=jax.ShapeDtypeStruct((NC, NP, D), jnp.float32),
        mesh=mesh,
        scratch_types=[
            pltpu.VMEM((G, C), jnp.int32),
            pltpu.VMEM((G, C), jnp.int32),
            pltpu.VMEM((C, D), jnp.float32),
            pltpu.VMEM((C, D), jnp.float32),
            pltpu.VMEM_SHARED((N, D), jnp.float32),
            pltpu.SemaphoreType.DMA,
            pltpu.SemaphoreType.DMA,
        ],
    )


def _sc_deg_body(dst_hbm, out_hbm, dstb, hist, idb, acc_sh):
    cid = lax.axis_index("c")
    sid = lax.axis_index("s")
    wid = sid * NC + cid

    zrow = jnp.zeros((16,), jnp.float32)
    ones = jnp.ones((16,), jnp.float32)
    iota = lax.iota(jnp.int32, 16)

    @pl.loop(0, HR)
    def _(i):
        @pl.loop(0, D // 16)
        def _(j):
            hist[i, pl.ds(j * 16, 16)] = zrow

    @pl.loop(0, HR // 16)
    def _(k):
        idb[0, pl.ds(k * 16, 16)] = iota + k * 16

    @pl.loop(sid, HR // 8, step=NS)
    def _(k):
        pltpu.sync_copy(hist.at[pl.ds(0, 8)], acc_sh.at[pl.ds(k * 8, 8)])

    pltpu.sync_copy(dst_hbm.at[wid], dstb)

    plsc.subcore_barrier()

    @pl.loop(0, EW // 16)
    def _(j):
        v = dstb[j]
        hi = lax.shift_right_logical(v, 7)
        lo = lax.bitwise_and(v, 127)
        plsc.addupdate_scatter(hist, [hi, lo], ones)

    pltpu.sync_copy(hist, acc_sh.at[idb.at[0]], add=True)

    plsc.subcore_barrier()

    @pl.loop(sid, HR // 8, step=NS)
    def _(k):
        pltpu.sync_copy(acc_sh.at[pl.ds(k * 8, 8)],
                        out_hbm.at[cid, pl.ds(k * 8, 8)])


@functools.cache
def _make_sc_deg():
    mesh = plsc.VectorSubcoreMesh(core_axis_name="c", subcore_axis_name="s")
    cp = pltpu.CompilerParams()
    if "needs_layout_passes" in pltpu.CompilerParams.__dataclass_fields__:
        cp = dataclasses.replace(cp, needs_layout_passes=False)
    return pl.kernel(
        _sc_deg_body,
        compiler_params=cp,
        out_type=jax.ShapeDtypeStruct((NC, HR, D), jnp.float32),
        mesh=mesh,
        scratch_types=[
            pltpu.VMEM((EW // 16, 16), jnp.int32),
            pltpu.VMEM((HR, D), jnp.float32),
            pltpu.VMEM((1, HR), jnp.int32),
            pltpu.VMEM_SHARED((HR, D), jnp.float32),
        ],
    )


ROWS_BLK = 1024


def _mm_body(x_ref, w_ref, o_ref):
    o_ref[...] = jnp.dot(x_ref[...], w_ref[...],
                         preferred_element_type=jnp.float32)


def _tc_matmul(x, w):
    return pl.pallas_call(
        _mm_body,
        grid=(NP // ROWS_BLK,),
        in_specs=[
            pl.BlockSpec((ROWS_BLK, D), lambda i: (i, 0)),
            pl.BlockSpec((D, D), lambda i: (0, 0)),
        ],
        out_specs=pl.BlockSpec((ROWS_BLK, D), lambda i: (i, 0)),
        out_shape=jax.ShapeDtypeStruct((NP, D), jnp.float32),
    )(x, w)


def _fuse1_body(s_ref, degp_ref, y_ref, b_ref, g_ref, be_ref, w_ref, o_ref):
    s = s_ref[0] + s_ref[1] + y_ref[...]
    deg = degp_ref[0] + degp_ref[1]
    h = s / (deg + 1.0) + b_ref[...]
    mean = jnp.mean(h, axis=-1, keepdims=True)
    var = jnp.mean((h - mean) ** 2, axis=-1, keepdims=True)
    h = (h - mean) * lax.rsqrt(var + 1e-5) * g_ref[...] + be_ref[...]
    h = jnp.maximum(h, 0.0)
    o_ref[...] = jnp.dot(h, w_ref[...], preferred_element_type=jnp.float32)


def _tc_fuse1(s0, degp, y0, b0, gamma0, beta0, W1):
    return pl.pallas_call(
        _fuse1_body,
        grid=(NP // ROWS_BLK,),
        in_specs=[
            pl.BlockSpec((NC, ROWS_BLK, D), lambda i: (0, i, 0)),
            pl.BlockSpec((NC, ROWS_BLK, 1), lambda i: (0, i, 0)),
            pl.BlockSpec((ROWS_BLK, D), lambda i: (i, 0)),
            pl.BlockSpec((1, D), lambda i: (0, 0)),
            pl.BlockSpec((1, D), lambda i: (0, 0)),
            pl.BlockSpec((1, D), lambda i: (0, 0)),
            pl.BlockSpec((D, D), lambda i: (0, 0)),
        ],
        out_specs=pl.BlockSpec((ROWS_BLK, D), lambda i: (i, 0)),
        out_shape=jax.ShapeDtypeStruct((NP, D), jnp.float32),
    )(s0, degp, y0, b0.reshape(1, D), gamma0.reshape(1, D),
      beta0.reshape(1, D), W1)


def _fuse2_body(s_ref, degp_ref, y_ref, b_ref, o_ref):
    s = s_ref[0] + s_ref[1] + y_ref[...]
    deg = degp_ref[0] + degp_ref[1]
    o_ref[...] = s / (deg + 1.0) + b_ref[...]


def _tc_fuse2(s1, degp, y1, b1):
    return pl.pallas_call(
        _fuse2_body,
        grid=(NP // ROWS_BLK,),
        in_specs=[
            pl.BlockSpec((NC, ROWS_BLK, D), lambda i: (0, i, 0)),
            pl.BlockSpec((NC, ROWS_BLK, 1), lambda i: (0, i, 0)),
            pl.BlockSpec((ROWS_BLK, D), lambda i: (i, 0)),
            pl.BlockSpec((1, D), lambda i: (0, 0)),
        ],
        out_specs=pl.BlockSpec((ROWS_BLK, D), lambda i: (i, 0)),
        out_shape=jax.ShapeDtypeStruct((NP, D), jnp.float32),
    )(s1, degp, y1, b1.reshape(1, D))


def kernel(feats, edge_index, W0, b0, gamma0, beta0, W1, b1):
    src = edge_index[0].reshape(NW, NBLK, G, C)
    dst = edge_index[1].reshape(NW, NBLK, G, C)
    dst16 = edge_index[1].reshape(NW, EW // 16, 16)
    featsp = jnp.concatenate(
        [feats, jnp.zeros((NP - N, D), jnp.float32)], axis=0)

    degp = _make_sc_deg()(dst16)
    degp = degp.reshape(NC, NP, 1)
    y0 = _tc_matmul(featsp, W0)
    s0 = _make_sc_segsum()(y0, src, dst)
    y1 = _tc_fuse1(s0, degp, y0, b0, gamma0, beta0, W1)
    s1 = _make_sc_segsum()(y1, src, dst)
    return _tc_fuse2(s1, degp, y1, b1)[:N]
